# Initial kernel scaffold; baseline (speedup 1.0000x reference)
import jax
import jax.numpy as jnp
from jax.experimental import pallas as pl


def kernel(x, edge_index, params):
    raise NotImplementedError("write your pallas kernel here")


# speedup vs baseline: 4.7656x; 4.7656x over previous
import functools

import jax
import jax.numpy as jnp
from jax import lax
from jax.experimental import pallas as pl
from jax.experimental.pallas import tpu as pltpu
from jax.experimental.pallas import tpu_sc as plsc

NC = 2
NS = 16
L = 16
CH = 128


def _cdiv(a, b):
    return (a + b - 1) // b


def _dot(a, b):
    return jnp.dot(a, b, preferred_element_type=jnp.float32,
                   precision=lax.Precision.HIGHEST)


def _prep_body(x_ref, wi, bi, a1, b1, bb1, h_ref, p_ref, q_ref):
    h = _dot(x_ref[...], wi[...]) + bi[...]
    h_ref[...] = h
    p_ref[...] = _dot(h, a1[...]) + bb1[...]
    q_ref[...] = _dot(h, b1[...])


def _update_mid_body(h_ref, t2, dg, mw2, b2, ua, ub, ub1, uw2, ub2,
                     na, nb, nb1, h_out, p_out, q_out):
    t = t2[0] + t2[1]
    agg = _dot(t, mw2[...]) + (dg[0] + dg[1]) * b2[...]
    h = h_ref[...]
    u1 = jnp.maximum(_dot(h, ua[...]) + _dot(agg, ub[...]) + ub1[...], 0.0)
    hn = jnp.maximum(_dot(u1, uw2[...]) + ub2[...], 0.0)
    h_out[...] = hn
    p_out[...] = _dot(hn, na[...]) + nb1[...]
    q_out[...] = _dot(hn, nb[...])


def _update_final_body(h_ref, t2, dg, mw2, b2, ua, ub, ub1, uw2, ub2,
                       wo, bo, o_out):
    t = t2[0] + t2[1]
    agg = _dot(t, mw2[...]) + (dg[0] + dg[1]) * b2[...]
    h = h_ref[...]
    u1 = jnp.maximum(_dot(h, ua[...]) + _dot(agg, ub[...]) + ub1[...], 0.0)
    hn = jnp.maximum(_dot(u1, uw2[...]) + ub2[...], 0.0)
    o_out[...] = _dot(hn, wo[...]) + bo[...]


def _row_spec(br, d):
    return pl.BlockSpec((br, d), lambda i: (i, 0))


def _full_spec(r, c):
    return pl.BlockSpec((r, c), lambda i: (0, 0))


def _pair_spec(br, d):
    return pl.BlockSpec((2, br, d), lambda i: (0, i, 0))


@functools.partial(jax.jit, static_argnums=(0, 1, 2))
def _tc_prep(n2, d, br, x, wi, bi, a1, b1, bb1):
    out = jax.ShapeDtypeStruct((n2, d), jnp.float32)
    return pl.pallas_call(
        _prep_body,
        grid=(n2 // br,),
        in_specs=[_row_spec(br, d), _full_spec(d, d), _full_spec(1, d),
                  _full_spec(d, d), _full_spec(d, d), _full_spec(1, d)],
        out_specs=[_row_spec(br, d)] * 3,
        out_shape=[out] * 3,
    )(x, wi, bi, a1, b1, bb1)


@functools.partial(jax.jit, static_argnums=(0, 1, 2))
def _tc_update_mid(n2, d, br, h, t2, dg, mw2, b2, ua, ub, ub1, uw2, ub2,
                   na, nb, nb1):
    out = jax.ShapeDtypeStruct((n2, d), jnp.float32)
    return pl.pallas_call(
        _update_mid_body,
        grid=(n2 // br,),
        in_specs=[_row_spec(br, d), _pair_spec(br, d), _pair_spec(br, d),
                  _full_spec(d, d), _full_spec(1, d),
                  _full_spec(d, d), _full_spec(d, d), _full_spec(1, d),
                  _full_spec(d, d), _full_spec(1, d),
                  _full_spec(d, d), _full_spec(d, d), _full_spec(1, d)],
        out_specs=[_row_spec(br, d)] * 3,
        out_shape=[out] * 3,
    )(h, t2, dg, mw2, b2, ua, ub, ub1, uw2, ub2, na, nb, nb1)


@functools.partial(jax.jit, static_argnums=(0, 1, 2))
def _tc_update_final(n2, d, br, h, t2, dg, mw2, b2, ua, ub, ub1, uw2, ub2,
                     wo, bo):
    out = jax.ShapeDtypeStruct((n2, d), jnp.float32)
    return pl.pallas_call(
        _update_final_body,
        grid=(n2 // br,),
        in_specs=[_row_spec(br, d), _pair_spec(br, d), _pair_spec(br, d),
                  _full_spec(d, d), _full_spec(1, d),
                  _full_spec(d, d), _full_spec(d, d), _full_spec(1, d),
                  _full_spec(d, d), _full_spec(1, d),
                  _full_spec(d, d), _full_spec(1, d)],
        out_specs=[_row_spec(br, d)],
        out_shape=[out],
    )(h, t2, dg, mw2, b2, ua, ub, ub1, uw2, ub2, wo, bo)[0]


def _sc_mesh():
    return plsc.VectorSubcoreMesh(core_axis_name="c", subcore_axis_name="s",
                                  num_cores=NC, num_subcores=NS)


@functools.partial(jax.jit, static_argnums=(0, 1, 2))
def _sc_edge_pass(n2, d, steps, p, q, row3, col3):
    kd = d // L
    rpt = n2 // NS
    nzb = rpt // CH

    @functools.partial(
        pl.kernel,
        out_type=jax.ShapeDtypeStruct((NC, n2, d), jnp.float32),
        mesh=_sc_mesh(),
        scratch_types=[
            pltpu.VMEM((CH,), jnp.int32),
            pltpu.VMEM((CH,), jnp.int32),
            pltpu.VMEM((CH, d), jnp.float32),
            pltpu.VMEM((CH, d), jnp.float32),
            pltpu.VMEM_SHARED((n2, d), jnp.float32),
            pltpu.SemaphoreType.DMA,
            pltpu.SemaphoreType.DMA,
        ],
    )
    def ek(p_hbm, q_hbm, row_hbm, col_hbm, out_hbm,
           row_v, col_v, pbuf, qbuf, t_sh, sem_p, sem_q):
        c = lax.axis_index("c")
        s = lax.axis_index("s")
        w = c * NS + s

        def zrow(i, carry):
            for k in range(kd):
                pbuf[i, pl.ds(k * L, L)] = jnp.zeros((L,), jnp.float32)
            return carry
        lax.fori_loop(0, CH, zrow, 0)

        def zt(i, carry):
            pltpu.sync_copy(pbuf, t_sh.at[pl.ds(s * rpt + i * CH, CH)])
            return carry
        lax.fori_loop(0, nzb, zt, 0)
        plsc.subcore_barrier()

        def step(j, carry):
            pltpu.sync_copy(row_hbm.at[w, j], row_v)
            pltpu.sync_copy(col_hbm.at[w, j], col_v)
            pltpu.async_copy(p_hbm.at[row_v], pbuf, sem_p)
            pltpu.async_copy(q_hbm.at[col_v], qbuf, sem_q)
            pltpu.make_async_copy(p_hbm.at[row_v], pbuf, sem_p).wait()
            pltpu.make_async_copy(q_hbm.at[col_v], qbuf, sem_q).wait()

            def crow(i, cc):
                for k in range(kd):
                    sl = pl.ds(k * L, L)
                    pbuf[i, sl] = jnp.maximum(pbuf[i, sl] + qbuf[i, sl], 0.0)
                return cc
            lax.fori_loop(0, CH, crow, 0)
            pltpu.sync_copy(pbuf, t_sh.at[col_v], add=True)
            return carry
        lax.fori_loop(0, steps, step, 0)
        plsc.subcore_barrier()

        pltpu.sync_copy(t_sh.at[pl.ds(s * rpt, rpt)],
                        out_hbm.at[c, pl.ds(s * rpt, rpt)])

    return ek(p, q, row3, col3)


@functools.partial(jax.jit, static_argnums=(0, 1, 2))
def _sc_degree(n2, d, steps, col3):
    kd = d // L
    rpt = n2 // NS
    nzb = rpt // CH

    @functools.partial(
        pl.kernel,
        out_type=jax.ShapeDtypeStruct((NC, n2, d), jnp.float32),
        mesh=_sc_mesh(),
        scratch_types=[
            pltpu.VMEM((CH,), jnp.int32),
            pltpu.VMEM((CH, d), jnp.float32),
            pltpu.VMEM_SHARED((n2, d), jnp.float32),
        ],
    )
    def dk(col_hbm, out_hbm, col_v, buf, d_sh):
        c = lax.axis_index("c")
        s = lax.axis_index("s")
        w = c * NS + s

        def fill(val):
            def frow(i, carry):
                for k in range(kd):
                    buf[i, pl.ds(k * L, L)] = jnp.full((L,), val, jnp.float32)
                return carry
            lax.fori_loop(0, CH, frow, 0)

        fill(0.0)

        def zt(i, carry):
            pltpu.sync_copy(buf, d_sh.at[pl.ds(s * rpt + i * CH, CH)])
            return carry
        lax.fori_loop(0, nzb, zt, 0)
        fill(1.0)
        plsc.subcore_barrier()

        def step(j, carry):
            pltpu.sync_copy(col_hbm.at[w, j], col_v)
            pltpu.sync_copy(buf, d_sh.at[col_v], add=True)
            return carry
        lax.fori_loop(0, steps, step, 0)
        plsc.subcore_barrier()

        pltpu.sync_copy(d_sh.at[pl.ds(s * rpt, rpt)],
                        out_hbm.at[c, pl.ds(s * rpt, rpt)])

    return dk(col3)


def kernel(x, edge_index, params):
    n, d = x.shape
    e = edge_index.shape[1]
    nw = NC * NS

    n2 = _cdiv(n, NS * CH) * NS * CH
    steps = _cdiv(_cdiv(e, nw), CH)
    e_pad = nw * steps * CH
    if e_pad > e and n2 == n:
        n2 += NS * CH
    br = 1024 if n2 % 1024 == 0 else n2 // NS

    x_pad = jnp.zeros((n2, d), jnp.float32).at[:n].set(x)

    n_spare = max(n2 - n, 1)
    pad_idx = (jnp.arange(e_pad - e, dtype=jnp.int32) % n_spare) + n
    row3 = jnp.concatenate([edge_index[0], pad_idx]).reshape(nw, steps, CH)
    col3 = jnp.concatenate([edge_index[1], pad_idx]).reshape(nw, steps, CH)

    prm = params
    lp = prm["layers"]

    def b_(v):
        return v.reshape(1, d)

    dg = _sc_degree(n2, d, steps, col3)

    h, p, q = _tc_prep(n2, d, br, x_pad, prm["Wi"], b_(prm["bi"]),
                       lp[0]["mW1"][:d], lp[0]["mW1"][d:], b_(lp[0]["mb1"]))

    for i in range(len(lp)):
        t2 = _sc_edge_pass(n2, d, steps, p, q, row3, col3)
        li = lp[i]
        common = (h, t2, dg, li["mW2"], b_(li["mb2"]),
                  li["uW1"][:d], li["uW1"][d:], b_(li["ub1"]),
                  li["uW2"], b_(li["ub2"]))
        if i + 1 < len(lp):
            ln = lp[i + 1]
            h, p, q = _tc_update_mid(n2, d, br, *common,
                                     ln["mW1"][:d], ln["mW1"][d:],
                                     b_(ln["mb1"]))
        else:
            out = _tc_update_final(n2, d, br, *common,
                                   prm["Wo"], b_(prm["bo"]))
    return out[:n]


# speedup vs baseline: 5.3097x; 1.1142x over previous
import functools

import jax
import jax.numpy as jnp
from jax import lax
from jax.experimental import pallas as pl
from jax.experimental.pallas import tpu as pltpu
from jax.experimental.pallas import tpu_sc as plsc

NC = 2
NS = 16
L = 16
CH = 80


def _cdiv(a, b):
    return (a + b - 1) // b


def _dot(a, b):
    return jnp.dot(a, b, preferred_element_type=jnp.float32,
                   precision=lax.Precision.HIGHEST)


def _prep_body(x_ref, wi, bi, a1, b1, bb1, h_ref, p_ref, q_ref):
    h = _dot(x_ref[...], wi[...]) + bi[...]
    h_ref[...] = h
    p_ref[...] = _dot(h, a1[...]) + bb1[...]
    q_ref[...] = _dot(h, b1[...])


def _update_mid_body(h_ref, t2, dg, mw2, b2, ua, ub, ub1, uw2, ub2,
                     na, nb, nb1, h_out, p_out, q_out):
    t = t2[0] + t2[1]
    agg = _dot(t, mw2[...]) + (dg[0] + dg[1]) * b2[...]
    h = h_ref[...]
    u1 = jnp.maximum(_dot(h, ua[...]) + _dot(agg, ub[...]) + ub1[...], 0.0)
    hn = jnp.maximum(_dot(u1, uw2[...]) + ub2[...], 0.0)
    h_out[...] = hn
    p_out[...] = _dot(hn, na[...]) + nb1[...]
    q_out[...] = _dot(hn, nb[...])


def _update_final_body(h_ref, t2, dg, mw2, b2, ua, ub, ub1, uw2, ub2,
                       wo, bo, o_out):
    t = t2[0] + t2[1]
    agg = _dot(t, mw2[...]) + (dg[0] + dg[1]) * b2[...]
    h = h_ref[...]
    u1 = jnp.maximum(_dot(h, ua[...]) + _dot(agg, ub[...]) + ub1[...], 0.0)
    hn = jnp.maximum(_dot(u1, uw2[...]) + ub2[...], 0.0)
    o_out[...] = _dot(hn, wo[...]) + bo[...]


def _row_spec(br, d):
    return pl.BlockSpec((br, d), lambda i: (i, 0))


def _full_spec(r, c):
    return pl.BlockSpec((r, c), lambda i: (0, 0))


def _pair_spec(br, d):
    return pl.BlockSpec((2, br, d), lambda i: (0, i, 0))


@functools.partial(jax.jit, static_argnums=(0, 1, 2))
def _tc_prep(n2, d, br, x, wi, bi, a1, b1, bb1):
    out = jax.ShapeDtypeStruct((n2, d), jnp.float32)
    return pl.pallas_call(
        _prep_body,
        grid=(n2 // br,),
        in_specs=[_row_spec(br, d), _full_spec(d, d), _full_spec(1, d),
                  _full_spec(d, d), _full_spec(d, d), _full_spec(1, d)],
        out_specs=[_row_spec(br, d)] * 3,
        out_shape=[out] * 3,
    )(x, wi, bi, a1, b1, bb1)


@functools.partial(jax.jit, static_argnums=(0, 1, 2))
def _tc_update_mid(n2, d, br, h, t2, dg, mw2, b2, ua, ub, ub1, uw2, ub2,
                   na, nb, nb1):
    out = jax.ShapeDtypeStruct((n2, d), jnp.float32)
    return pl.pallas_call(
        _update_mid_body,
        grid=(n2 // br,),
        in_specs=[_row_spec(br, d), _pair_spec(br, d), _pair_spec(br, d),
                  _full_spec(d, d), _full_spec(1, d),
                  _full_spec(d, d), _full_spec(d, d), _full_spec(1, d),
                  _full_spec(d, d), _full_spec(1, d),
                  _full_spec(d, d), _full_spec(d, d), _full_spec(1, d)],
        out_specs=[_row_spec(br, d)] * 3,
        out_shape=[out] * 3,
    )(h, t2, dg, mw2, b2, ua, ub, ub1, uw2, ub2, na, nb, nb1)


@functools.partial(jax.jit, static_argnums=(0, 1, 2))
def _tc_update_final(n2, d, br, h, t2, dg, mw2, b2, ua, ub, ub1, uw2, ub2,
                     wo, bo):
    out = jax.ShapeDtypeStruct((n2, d), jnp.float32)
    return pl.pallas_call(
        _update_final_body,
        grid=(n2 // br,),
        in_specs=[_row_spec(br, d), _pair_spec(br, d), _pair_spec(br, d),
                  _full_spec(d, d), _full_spec(1, d),
                  _full_spec(d, d), _full_spec(d, d), _full_spec(1, d),
                  _full_spec(d, d), _full_spec(1, d),
                  _full_spec(d, d), _full_spec(1, d)],
        out_specs=[_row_spec(br, d)],
        out_shape=[out],
    )(h, t2, dg, mw2, b2, ua, ub, ub1, uw2, ub2, wo, bo)[0]


def _sc_mesh():
    return plsc.VectorSubcoreMesh(core_axis_name="c", subcore_axis_name="s",
                                  num_cores=NC, num_subcores=NS)


@functools.partial(jax.jit, static_argnums=(0, 1, 2))
def _sc_edge_pass(n2, d, steps, p, q, row3, col3):
    kd = d // L
    rpt = n2 // NS
    nzb = rpt // CH

    @functools.partial(
        pl.kernel,
        out_type=jax.ShapeDtypeStruct((NC, n2, d), jnp.float32),
        mesh=_sc_mesh(),
        scratch_types=[
            pltpu.VMEM((2, CH), jnp.int32),
            pltpu.VMEM((2, CH), jnp.int32),
            pltpu.VMEM((2, CH, d), jnp.float32),
            pltpu.VMEM((2, CH, d), jnp.float32),
            pltpu.VMEM_SHARED((n2, d), jnp.float32),
            pltpu.SemaphoreType.DMA,
            pltpu.SemaphoreType.DMA,
            pltpu.SemaphoreType.DMA,
        ],
    )
    def ek(p_hbm, q_hbm, row_hbm, col_hbm, out_hbm,
           row_v, col_v, pbuf, qbuf, t_sh, sem_p, sem_q, sem_s):
        c = lax.axis_index("c")
        s = lax.axis_index("s")
        w = c * NS + s

        def zrow(i, carry):
            for k in range(kd):
                pbuf[0, i, pl.ds(k * L, L)] = jnp.zeros((L,), jnp.float32)
            return carry
        lax.fori_loop(0, CH, zrow, 0)

        def zt(i, carry):
            pltpu.sync_copy(pbuf.at[0], t_sh.at[pl.ds(s * rpt + i * CH, CH)])
            return carry
        lax.fori_loop(0, nzb, zt, 0)
        plsc.subcore_barrier()

        pltpu.sync_copy(row_hbm.at[w, 0], row_v.at[0])
        pltpu.sync_copy(col_hbm.at[w, 0], col_v.at[0])
        pltpu.async_copy(p_hbm.at[row_v.at[0]], pbuf.at[0], sem_p)
        pltpu.async_copy(q_hbm.at[col_v.at[0]], qbuf.at[0], sem_q)

        def pair(jj, carry):
            for b in range(2):
                j = jj * 2 + b
                nb = 1 - b
                pltpu.make_async_copy(p_hbm.at[row_v.at[b]], pbuf.at[b],
                                      sem_p).wait()
                pltpu.make_async_copy(q_hbm.at[col_v.at[b]], qbuf.at[b],
                                      sem_q).wait()

                @pl.when(j + 1 < steps)
                def _prefetch():
                    @pl.when(j >= 1)
                    def _drain_prev_scatter():
                        pltpu.make_async_copy(
                            pbuf.at[nb], t_sh.at[col_v.at[nb]], sem_s).wait()
                    pltpu.sync_copy(row_hbm.at[w, j + 1], row_v.at[nb])
                    pltpu.sync_copy(col_hbm.at[w, j + 1], col_v.at[nb])
                    pltpu.async_copy(p_hbm.at[row_v.at[nb]], pbuf.at[nb],
                                     sem_p)
                    pltpu.async_copy(q_hbm.at[col_v.at[nb]], qbuf.at[nb],
                                     sem_q)

                def crow(i, cc):
                    for k in range(kd):
                        sl = pl.ds(k * L, L)
                        pbuf[b, i, sl] = jnp.maximum(
                            pbuf[b, i, sl] + qbuf[b, i, sl], 0.0)
                    return cc
                lax.fori_loop(0, CH, crow, 0)
                pltpu.async_copy(pbuf.at[b], t_sh.at[col_v.at[b]], sem_s,
                                 add=True)
            return carry
        lax.fori_loop(0, steps // 2, pair, 0)
        pltpu.make_async_copy(pbuf.at[0], t_sh.at[col_v.at[0]], sem_s).wait()
        pltpu.make_async_copy(pbuf.at[1], t_sh.at[col_v.at[1]], sem_s).wait()
        plsc.subcore_barrier()

        pltpu.sync_copy(t_sh.at[pl.ds(s * rpt, rpt)],
                        out_hbm.at[c, pl.ds(s * rpt, rpt)])

    return ek(p, q, row3, col3)


@functools.partial(jax.jit, static_argnums=(0, 1, 2))
def _sc_degree(n2, d, steps, col3):
    kd = d // L
    rpt = n2 // NS
    nzb = rpt // CH

    @functools.partial(
        pl.kernel,
        out_type=jax.ShapeDtypeStruct((NC, n2, d), jnp.float32),
        mesh=_sc_mesh(),
        scratch_types=[
            pltpu.VMEM((2, CH), jnp.int32),
            pltpu.VMEM((CH, d), jnp.float32),
            pltpu.VMEM_SHARED((n2, d), jnp.float32),
            pltpu.SemaphoreType.DMA,
        ],
    )
    def dk(col_hbm, out_hbm, col_v, buf, d_sh, sem_s):
        c = lax.axis_index("c")
        s = lax.axis_index("s")
        w = c * NS + s

        def fill(val):
            def frow(i, carry):
                for k in range(kd):
                    buf[i, pl.ds(k * L, L)] = jnp.full((L,), val, jnp.float32)
                return carry
            lax.fori_loop(0, CH, frow, 0)

        fill(0.0)

        def zt(i, carry):
            pltpu.sync_copy(buf, d_sh.at[pl.ds(s * rpt + i * CH, CH)])
            return carry
        lax.fori_loop(0, nzb, zt, 0)
        fill(1.0)
        plsc.subcore_barrier()

        pltpu.sync_copy(col_hbm.at[w, 0], col_v.at[0])

        def pair(jj, carry):
            for b in range(2):
                j = jj * 2 + b
                nb = 1 - b

                @pl.when(j + 1 < steps)
                def _prefetch():
                    @pl.when(j >= 1)
                    def _drain_prev():
                        pltpu.make_async_copy(
                            buf, d_sh.at[col_v.at[nb]], sem_s).wait()
                    pltpu.sync_copy(col_hbm.at[w, j + 1], col_v.at[nb])
                pltpu.async_copy(buf, d_sh.at[col_v.at[b]], sem_s, add=True)
            return carry
        lax.fori_loop(0, steps // 2, pair, 0)
        pltpu.make_async_copy(buf, d_sh.at[col_v.at[0]], sem_s).wait()
        pltpu.make_async_copy(buf, d_sh.at[col_v.at[1]], sem_s).wait()
        plsc.subcore_barrier()

        pltpu.sync_copy(d_sh.at[pl.ds(s * rpt, rpt)],
                        out_hbm.at[c, pl.ds(s * rpt, rpt)])

    return dk(col3)


def kernel(x, edge_index, params):
    n, d = x.shape
    e = edge_index.shape[1]
    nw = NC * NS

    n2 = _cdiv(n, NS * CH) * NS * CH
    steps = _cdiv(_cdiv(e, nw), CH)
    steps += steps % 2
    e_pad = nw * steps * CH
    if e_pad > e and n2 == n:
        n2 += NS * CH
    br = 1024 if n2 % 1024 == 0 else n2 // NS

    x_pad = jnp.zeros((n2, d), jnp.float32).at[:n].set(x)

    n_spare = max(n2 - n, 1)
    pad_idx = (jnp.arange(e_pad - e, dtype=jnp.int32) % n_spare) + n
    row3 = jnp.concatenate([edge_index[0], pad_idx]).reshape(nw, steps, CH)
    col3 = jnp.concatenate([edge_index[1], pad_idx]).reshape(nw, steps, CH)

    prm = params
    lp = prm["layers"]

    def b_(v):
        return v.reshape(1, d)

    dg = _sc_degree(n2, d, steps, col3)

    h, p, q = _tc_prep(n2, d, br, x_pad, prm["Wi"], b_(prm["bi"]),
                       lp[0]["mW1"][:d], lp[0]["mW1"][d:], b_(lp[0]["mb1"]))

    for i in range(len(lp)):
        t2 = _sc_edge_pass(n2, d, steps, p, q, row3, col3)
        li = lp[i]
        common = (h, t2, dg, li["mW2"], b_(li["mb2"]),
                  li["uW1"][:d], li["uW1"][d:], b_(li["ub1"]),
                  li["uW2"], b_(li["ub2"]))
        if i + 1 < len(lp):
            ln = lp[i + 1]
            h, p, q = _tc_update_mid(n2, d, br, *common,
                                     ln["mW1"][:d], ln["mW1"][d:],
                                     b_(ln["mb1"]))
        else:
            out = _tc_update_final(n2, d, br, *common,
                                   prm["Wo"], b_(prm["bo"]))
    return out[:n]


# speedup vs baseline: 8.1104x; 1.5275x over previous
import functools

import jax
import jax.numpy as jnp
from jax import lax
from jax.experimental import pallas as pl
from jax.experimental.pallas import tpu as pltpu
from jax.experimental.pallas import tpu_sc as plsc

NC = 2
NS = 16
L = 16
CH = 80
SB = 16


def _cdiv(a, b):
    return (a + b - 1) // b


def _dot(a, b):
    return jnp.dot(a, b, preferred_element_type=jnp.float32,
                   precision=lax.Precision.HIGHEST)


def _prep_body(x_ref, wi, bi, a1, b1, bb1, h_ref, p_ref, q_ref):
    h = _dot(x_ref[...], wi[...]) + bi[...]
    h_ref[...] = h
    p_ref[...] = _dot(h, a1[...]) + bb1[...]
    q_ref[...] = _dot(h, b1[...])


def _update_mid_body(h_ref, t2, dg, mw2, b2, ua, ub, ub1, uw2, ub2,
                     na, nb, nb1, h_out, p_out, q_out):
    t = t2[0] + t2[1]
    agg = _dot(t, mw2[...]) + (dg[0] + dg[1]) * b2[...]
    h = h_ref[...]
    u1 = jnp.maximum(_dot(h, ua[...]) + _dot(agg, ub[...]) + ub1[...], 0.0)
    hn = jnp.maximum(_dot(u1, uw2[...]) + ub2[...], 0.0)
    h_out[...] = hn
    p_out[...] = _dot(hn, na[...]) + nb1[...]
    q_out[...] = _dot(hn, nb[...])


def _update_final_body(h_ref, t2, dg, mw2, b2, ua, ub, ub1, uw2, ub2,
                       wo, bo, o_out):
    t = t2[0] + t2[1]
    agg = _dot(t, mw2[...]) + (dg[0] + dg[1]) * b2[...]
    h = h_ref[...]
    u1 = jnp.maximum(_dot(h, ua[...]) + _dot(agg, ub[...]) + ub1[...], 0.0)
    hn = jnp.maximum(_dot(u1, uw2[...]) + ub2[...], 0.0)
    o_out[...] = _dot(hn, wo[...]) + bo[...]


def _row_spec(br, d):
    return pl.BlockSpec((br, d), lambda i: (i, 0))


def _full_spec(r, c):
    return pl.BlockSpec((r, c), lambda i: (0, 0))


def _pair_spec(br, d):
    return pl.BlockSpec((2, br, d), lambda i: (0, i, 0))


@functools.partial(jax.jit, static_argnums=(0, 1, 2))
def _tc_prep(n2, d, br, x, wi, bi, a1, b1, bb1):
    out = jax.ShapeDtypeStruct((n2, d), jnp.float32)
    return pl.pallas_call(
        _prep_body,
        grid=(n2 // br,),
        in_specs=[_row_spec(br, d), _full_spec(d, d), _full_spec(1, d),
                  _full_spec(d, d), _full_spec(d, d), _full_spec(1, d)],
        out_specs=[_row_spec(br, d)] * 3,
        out_shape=[out] * 3,
    )(x, wi, bi, a1, b1, bb1)


@functools.partial(jax.jit, static_argnums=(0, 1, 2))
def _tc_update_mid(n2, d, br, h, t2, dg, mw2, b2, ua, ub, ub1, uw2, ub2,
                   na, nb, nb1):
    out = jax.ShapeDtypeStruct((n2, d), jnp.float32)
    return pl.pallas_call(
        _update_mid_body,
        grid=(n2 // br,),
        in_specs=[_row_spec(br, d), _pair_spec(br, d), _pair_spec(br, d),
                  _full_spec(d, d), _full_spec(1, d),
                  _full_spec(d, d), _full_spec(d, d), _full_spec(1, d),
                  _full_spec(d, d), _full_spec(1, d),
                  _full_spec(d, d), _full_spec(d, d), _full_spec(1, d)],
        out_specs=[_row_spec(br, d)] * 3,
        out_shape=[out] * 3,
    )(h, t2, dg, mw2, b2, ua, ub, ub1, uw2, ub2, na, nb, nb1)


@functools.partial(jax.jit, static_argnums=(0, 1, 2))
def _tc_update_final(n2, d, br, h, t2, dg, mw2, b2, ua, ub, ub1, uw2, ub2,
                     wo, bo):
    out = jax.ShapeDtypeStruct((n2, d), jnp.float32)
    return pl.pallas_call(
        _update_final_body,
        grid=(n2 // br,),
        in_specs=[_row_spec(br, d), _pair_spec(br, d), _pair_spec(br, d),
                  _full_spec(d, d), _full_spec(1, d),
                  _full_spec(d, d), _full_spec(d, d), _full_spec(1, d),
                  _full_spec(d, d), _full_spec(1, d),
                  _full_spec(d, d), _full_spec(1, d)],
        out_specs=[_row_spec(br, d)],
        out_shape=[out],
    )(h, t2, dg, mw2, b2, ua, ub, ub1, uw2, ub2, wo, bo)[0]


def _sc_mesh():
    return plsc.VectorSubcoreMesh(core_axis_name="c", subcore_axis_name="s",
                                  num_cores=NC, num_subcores=NS)


@functools.partial(jax.jit, static_argnums=(0, 1, 2))
def _sc_edge_pass(n2, d, steps, p, q, row3, col3):
    kd = d // L
    rpt = n2 // NS
    nzb = rpt // CH
    nsb = steps // SB

    @functools.partial(
        pl.kernel,
        out_type=jax.ShapeDtypeStruct((NC, n2, d), jnp.float32),
        mesh=_sc_mesh(),
        scratch_types=[
            pltpu.VMEM((2, SB, CH), jnp.int32),
            pltpu.VMEM((2, SB, CH), jnp.int32),
            pltpu.VMEM((2, CH, d), jnp.float32),
            pltpu.VMEM((2, CH, d), jnp.float32),
            pltpu.VMEM_SHARED((n2, d), jnp.float32),
            pltpu.SemaphoreType.DMA,
            [pltpu.SemaphoreType.DMA] * 2,
            pltpu.SemaphoreType.DMA,
        ],
    )
    def ek(p_hbm, q_hbm, row_hbm, col_hbm, out_hbm,
           row_v, col_v, pbuf, qbuf, t_sh, sem_i, sem_g, sem_s):
        c = lax.axis_index("c")
        s = lax.axis_index("s")
        w = c * NS + s

        def zrow(i, carry):
            for k in range(kd):
                pbuf[0, i, pl.ds(k * L, L)] = jnp.zeros((L,), jnp.float32)
            return carry
        lax.fori_loop(0, CH, zrow, 0)

        def zt(i, carry):
            pltpu.sync_copy(pbuf.at[0], t_sh.at[pl.ds(s * rpt + i * CH, CH)])
            return carry
        lax.fori_loop(0, nzb, zt, 0)
        plsc.subcore_barrier()

        def issue_idx(slot, sb):
            sl = pl.ds(sb * SB, SB)
            pltpu.async_copy(row_hbm.at[w, sl], row_v.at[slot], sem_i)
            pltpu.async_copy(col_hbm.at[w, sl], col_v.at[slot], sem_i)

        def wait_idx(slot):
            pltpu.make_async_copy(row_hbm.at[w, pl.ds(0, SB)],
                                  row_v.at[slot], sem_i).wait()
            pltpu.make_async_copy(col_hbm.at[w, pl.ds(0, SB)],
                                  col_v.at[slot], sem_i).wait()

        def issue_gather(slot, b, r):
            pltpu.async_copy(p_hbm.at[row_v.at[slot, b]], pbuf.at[r],
                             sem_g[r])
            pltpu.async_copy(q_hbm.at[col_v.at[slot, b]], qbuf.at[r],
                             sem_g[r])

        def wait_gather(r):
            pltpu.make_async_copy(p_hbm.at[row_v.at[0, 0]], pbuf.at[r],
                                  sem_g[r]).wait()
            pltpu.make_async_copy(q_hbm.at[col_v.at[0, 0]], qbuf.at[r],
                                  sem_g[r]).wait()

        def drain_scatter(r):
            pltpu.make_async_copy(pbuf.at[r], t_sh.at[col_v.at[0, 0]],
                                  sem_s).wait()

        issue_idx(0, 0)
        wait_idx(0)
        issue_gather(0, 0, 0)

        def sb_body(sb, carry):
            slot = sb % 2
            nslot = 1 - slot

            @pl.when(sb + 1 < nsb)
            def _prefetch_idx():
                issue_idx(nslot, sb + 1)

            for b in range(SB):
                r = b % 2
                nr = 1 - r
                if b == 0:
                    @pl.when(sb >= 1)
                    def _drain0():
                        drain_scatter(nr)
                else:
                    drain_scatter(nr)
                if b + 1 < SB:
                    issue_gather(slot, b + 1, nr)
                else:
                    @pl.when(sb + 1 < nsb)
                    def _tail_gather():
                        wait_idx(nslot)
                        issue_gather(nslot, 0, nr)

                wait_gather(r)

                def crow(i, cc):
                    for k in range(kd):
                        sl = pl.ds(k * L, L)
                        pbuf[r, i, sl] = jnp.maximum(
                            pbuf[r, i, sl] + qbuf[r, i, sl], 0.0)
                    return cc
                lax.fori_loop(0, CH, crow, 0)
                pltpu.async_copy(pbuf.at[r], t_sh.at[col_v.at[slot, b]],
                                 sem_s, add=True)
            return carry
        lax.fori_loop(0, nsb, sb_body, 0)
        drain_scatter((SB - 1) % 2)
        plsc.subcore_barrier()

        pltpu.sync_copy(t_sh.at[pl.ds(s * rpt, rpt)],
                        out_hbm.at[c, pl.ds(s * rpt, rpt)])

    return ek(p, q, row3, col3)


@functools.partial(jax.jit, static_argnums=(0, 1, 2))
def _sc_degree(n2, d, steps, col3):
    kd = d // L
    rpt = n2 // NS
    nzb = rpt // CH
    nsb = steps // SB

    @functools.partial(
        pl.kernel,
        out_type=jax.ShapeDtypeStruct((NC, n2, d), jnp.float32),
        mesh=_sc_mesh(),
        scratch_types=[
            pltpu.VMEM((2, SB, CH), jnp.int32),
            pltpu.VMEM((CH, d), jnp.float32),
            pltpu.VMEM_SHARED((n2, d), jnp.float32),
            pltpu.SemaphoreType.DMA,
            pltpu.SemaphoreType.DMA,
        ],
    )
    def dk(col_hbm, out_hbm, col_v, buf, d_sh, sem_i, sem_s):
        c = lax.axis_index("c")
        s = lax.axis_index("s")
        w = c * NS + s

        def fill(val):
            def frow(i, carry):
                for k in range(kd):
                    buf[i, pl.ds(k * L, L)] = jnp.full((L,), val, jnp.float32)
                return carry
            lax.fori_loop(0, CH, frow, 0)

        fill(0.0)

        def zt(i, carry):
            pltpu.sync_copy(buf, d_sh.at[pl.ds(s * rpt + i * CH, CH)])
            return carry
        lax.fori_loop(0, nzb, zt, 0)
        fill(1.0)
        plsc.subcore_barrier()

        def issue_idx(slot, sb):
            pltpu.async_copy(col_hbm.at[w, pl.ds(sb * SB, SB)],
                             col_v.at[slot], sem_i)

        def wait_idx(slot):
            pltpu.make_async_copy(col_hbm.at[w, pl.ds(0, SB)],
                                  col_v.at[slot], sem_i).wait()

        issue_idx(0, 0)
        wait_idx(0)

        def sb_body(sb, carry):
            slot = sb % 2
            nslot = 1 - slot

            @pl.when(sb + 1 < nsb)
            def _prefetch_idx():
                issue_idx(nslot, sb + 1)

            for b in range(SB):
                if b >= 2:
                    pltpu.make_async_copy(
                        buf, d_sh.at[col_v.at[0, 0]], sem_s).wait()
                else:
                    @pl.when(sb >= 1)
                    def _drain0():
                        pltpu.make_async_copy(
                            buf, d_sh.at[col_v.at[0, 0]], sem_s).wait()
                if b == SB - 1:
                    @pl.when(sb + 1 < nsb)
                    def _wait_next_idx():
                        wait_idx(nslot)
                pltpu.async_copy(buf, d_sh.at[col_v.at[slot, b]], sem_s,
                                 add=True)
            return carry
        lax.fori_loop(0, nsb, sb_body, 0)
        pltpu.make_async_copy(buf, d_sh.at[col_v.at[0, 0]], sem_s).wait()
        pltpu.make_async_copy(buf, d_sh.at[col_v.at[0, 0]], sem_s).wait()
        plsc.subcore_barrier()

        pltpu.sync_copy(d_sh.at[pl.ds(s * rpt, rpt)],
                        out_hbm.at[c, pl.ds(s * rpt, rpt)])

    return dk(col3)


def kernel(x, edge_index, params):
    n, d = x.shape
    e = edge_index.shape[1]
    nw = NC * NS

    n2 = _cdiv(n, NS * CH) * NS * CH
    steps = _cdiv(_cdiv(e, nw), CH)
    steps = _cdiv(steps, SB) * SB
    e_pad = nw * steps * CH
    if e_pad > e and n2 == n:
        n2 += NS * CH
    br = 1024 if n2 % 1024 == 0 else n2 // NS

    x_pad = jnp.zeros((n2, d), jnp.float32).at[:n].set(x)

    n_spare = max(n2 - n, 1)
    pad_idx = (jnp.arange(e_pad - e, dtype=jnp.int32) % n_spare) + n
    row3 = jnp.concatenate([edge_index[0], pad_idx]).reshape(nw, steps, CH)
    col3 = jnp.concatenate([edge_index[1], pad_idx]).reshape(nw, steps, CH)

    prm = params
    lp = prm["layers"]

    def b_(v):
        return v.reshape(1, d)

    dg = _sc_degree(n2, d, steps, col3)

    h, p, q = _tc_prep(n2, d, br, x_pad, prm["Wi"], b_(prm["bi"]),
                       lp[0]["mW1"][:d], lp[0]["mW1"][d:], b_(lp[0]["mb1"]))

    for i in range(len(lp)):
        t2 = _sc_edge_pass(n2, d, steps, p, q, row3, col3)
        li = lp[i]
        common = (h, t2, dg, li["mW2"], b_(li["mb2"]),
                  li["uW1"][:d], li["uW1"][d:], b_(li["ub1"]),
                  li["uW2"], b_(li["ub2"]))
        if i + 1 < len(lp):
            ln = lp[i + 1]
            h, p, q = _tc_update_mid(n2, d, br, *common,
                                     ln["mW1"][:d], ln["mW1"][d:],
                                     b_(ln["mb1"]))
        else:
            out = _tc_update_final(n2, d, br, *common,
                                   prm["Wo"], b_(prm["bo"]))
    return out[:n]


# speedup vs baseline: 8.8124x; 1.0865x over previous
import functools

import jax
import jax.numpy as jnp
import numpy as np
from jax import lax
from jax.experimental import pallas as pl
from jax.experimental.pallas import tpu as pltpu
from jax.experimental.pallas import tpu_sc as plsc

NC = 2
NS = 16
L = 16
CH = 80
SB = 16


def _cdiv(a, b):
    return (a + b - 1) // b


def _dot(a, b):
    return jnp.dot(a, b, preferred_element_type=jnp.float32,
                   precision=lax.Precision.HIGHEST)


def _prep_body(x_ref, wi, bi, a1, b1, bb1, h_ref, p_ref, q_ref):
    h = _dot(x_ref[...], wi[...]) + bi[...]
    h_ref[...] = h
    p_ref[...] = _dot(h, a1[...]) + bb1[...]
    q_ref[...] = _dot(h, b1[...])


def _update_mid_body(h_ref, t2, dg, mw2, b2, ua, ub, ub1, uw2, ub2,
                     na, nb, nb1, h_out, p_out, q_out):
    t = t2[0] + t2[1]
    agg = _dot(t, mw2[...]) + (dg[0] + dg[1]) * b2[...]
    h = h_ref[...]
    u1 = jnp.maximum(_dot(h, ua[...]) + _dot(agg, ub[...]) + ub1[...], 0.0)
    hn = jnp.maximum(_dot(u1, uw2[...]) + ub2[...], 0.0)
    h_out[...] = hn
    p_out[...] = _dot(hn, na[...]) + nb1[...]
    q_out[...] = _dot(hn, nb[...])


def _update_final_body(h_ref, t2, dg, mw2, b2, ua, ub, ub1, uw2, ub2,
                       wo, bo, o_out):
    t = t2[0] + t2[1]
    agg = _dot(t, mw2[...]) + (dg[0] + dg[1]) * b2[...]
    h = h_ref[...]
    u1 = jnp.maximum(_dot(h, ua[...]) + _dot(agg, ub[...]) + ub1[...], 0.0)
    hn = jnp.maximum(_dot(u1, uw2[...]) + ub2[...], 0.0)
    o_out[...] = _dot(hn, wo[...]) + bo[...]


def _row_spec(br, d):
    return pl.BlockSpec((br, d), lambda i: (i, 0))


def _full_spec(r, c):
    return pl.BlockSpec((r, c), lambda i: (0, 0))


def _pair_spec(br, d):
    return pl.BlockSpec((2, br, d), lambda i: (0, i, 0))


@functools.partial(jax.jit, static_argnums=(0, 1, 2))
def _tc_prep(n2, d, br, x, wi, bi, a1, b1, bb1):
    out = jax.ShapeDtypeStruct((n2, d), jnp.float32)
    return pl.pallas_call(
        _prep_body,
        grid=(n2 // br,),
        in_specs=[_row_spec(br, d), _full_spec(d, d), _full_spec(1, d),
                  _full_spec(d, d), _full_spec(d, d), _full_spec(1, d)],
        out_specs=[_row_spec(br, d)] * 3,
        out_shape=[out] * 3,
    )(x, wi, bi, a1, b1, bb1)


@functools.partial(jax.jit, static_argnums=(0, 1, 2))
def _tc_update_mid(n2, d, br, h, t2, dg, mw2, b2, ua, ub, ub1, uw2, ub2,
                   na, nb, nb1):
    out = jax.ShapeDtypeStruct((n2, d), jnp.float32)
    return pl.pallas_call(
        _update_mid_body,
        grid=(n2 // br,),
        in_specs=[_row_spec(br, d), _pair_spec(br, d), _pair_spec(br, 1),
                  _full_spec(d, d), _full_spec(1, d),
                  _full_spec(d, d), _full_spec(d, d), _full_spec(1, d),
                  _full_spec(d, d), _full_spec(1, d),
                  _full_spec(d, d), _full_spec(d, d), _full_spec(1, d)],
        out_specs=[_row_spec(br, d)] * 3,
        out_shape=[out] * 3,
    )(h, t2, dg, mw2, b2, ua, ub, ub1, uw2, ub2, na, nb, nb1)


@functools.partial(jax.jit, static_argnums=(0, 1, 2))
def _tc_update_final(n2, d, br, h, t2, dg, mw2, b2, ua, ub, ub1, uw2, ub2,
                     wo, bo):
    out = jax.ShapeDtypeStruct((n2, d), jnp.float32)
    return pl.pallas_call(
        _update_final_body,
        grid=(n2 // br,),
        in_specs=[_row_spec(br, d), _pair_spec(br, d), _pair_spec(br, 1),
                  _full_spec(d, d), _full_spec(1, d),
                  _full_spec(d, d), _full_spec(d, d), _full_spec(1, d),
                  _full_spec(d, d), _full_spec(1, d),
                  _full_spec(d, d), _full_spec(1, d)],
        out_specs=[_row_spec(br, d)],
        out_shape=[out],
    )(h, t2, dg, mw2, b2, ua, ub, ub1, uw2, ub2, wo, bo)[0]


def _sc_mesh():
    return plsc.VectorSubcoreMesh(core_axis_name="c", subcore_axis_name="s",
                                  num_cores=NC, num_subcores=NS)


@functools.partial(jax.jit, static_argnums=(0, 1, 2))
def _sc_edge_pass(n2, d, steps, p, q, row3, col3):
    kd = d // L
    rpt = n2 // NS
    nzb = rpt // CH
    nsb = steps // SB

    @functools.partial(
        pl.kernel,
        out_type=jax.ShapeDtypeStruct((NC, n2, d), jnp.float32),
        mesh=_sc_mesh(),
        scratch_types=[
            pltpu.VMEM((2, SB, CH), jnp.int32),
            pltpu.VMEM((2, SB, CH), jnp.int32),
            pltpu.VMEM((2, CH, d), jnp.float32),
            pltpu.VMEM((2, CH, d), jnp.float32),
            pltpu.VMEM_SHARED((n2, d), jnp.float32),
            pltpu.SemaphoreType.DMA,
            [pltpu.SemaphoreType.DMA] * 2,
            pltpu.SemaphoreType.DMA,
        ],
    )
    def ek(p_hbm, q_hbm, row_hbm, col_hbm, out_hbm,
           row_v, col_v, pbuf, qbuf, t_sh, sem_i, sem_g, sem_s):
        c = lax.axis_index("c")
        s = lax.axis_index("s")
        w = c * NS + s

        def zrow(i, carry):
            for k in range(kd):
                pbuf[0, i, pl.ds(k * L, L)] = jnp.zeros((L,), jnp.float32)
            return carry
        lax.fori_loop(0, CH, zrow, 0)

        def zt(i, carry):
            pltpu.sync_copy(pbuf.at[0], t_sh.at[pl.ds(s * rpt + i * CH, CH)])
            return carry
        lax.fori_loop(0, nzb, zt, 0)
        plsc.subcore_barrier()

        def issue_idx(slot, sb):
            sl = pl.ds(sb * SB, SB)
            pltpu.async_copy(row_hbm.at[w, sl], row_v.at[slot], sem_i)
            pltpu.async_copy(col_hbm.at[w, sl], col_v.at[slot], sem_i)

        def wait_idx(slot):
            pltpu.make_async_copy(row_hbm.at[w, pl.ds(0, SB)],
                                  row_v.at[slot], sem_i).wait()
            pltpu.make_async_copy(col_hbm.at[w, pl.ds(0, SB)],
                                  col_v.at[slot], sem_i).wait()

        def issue_gather(slot, b, r):
            pltpu.async_copy(p_hbm.at[row_v.at[slot, b]], pbuf.at[r],
                             sem_g[r])
            pltpu.async_copy(q_hbm.at[col_v.at[slot, b]], qbuf.at[r],
                             sem_g[r])

        def wait_gather(r):
            pltpu.make_async_copy(p_hbm.at[row_v.at[0, 0]], pbuf.at[r],
                                  sem_g[r]).wait()
            pltpu.make_async_copy(q_hbm.at[col_v.at[0, 0]], qbuf.at[r],
                                  sem_g[r]).wait()

        def drain_scatter(r):
            pltpu.make_async_copy(pbuf.at[r], t_sh.at[col_v.at[0, 0]],
                                  sem_s).wait()

        issue_idx(0, 0)
        wait_idx(0)
        issue_gather(0, 0, 0)

        def sb_body(sb, carry):
            slot = sb % 2
            nslot = 1 - slot

            @pl.when(sb + 1 < nsb)
            def _prefetch_idx():
                issue_idx(nslot, sb + 1)

            for b in range(SB):
                r = b % 2
                nr = 1 - r
                if b == 0:
                    @pl.when(sb >= 1)
                    def _drain0():
                        drain_scatter(nr)
                else:
                    drain_scatter(nr)
                if b + 1 < SB:
                    issue_gather(slot, b + 1, nr)
                else:
                    @pl.when(sb + 1 < nsb)
                    def _tail_gather():
                        wait_idx(nslot)
                        issue_gather(nslot, 0, nr)

                wait_gather(r)

                def crow(i, cc):
                    for k in range(kd):
                        sl = pl.ds(k * L, L)
                        pbuf[r, i, sl] = jnp.maximum(
                            pbuf[r, i, sl] + qbuf[r, i, sl], 0.0)
                    return cc
                lax.fori_loop(0, CH, crow, 0)
                pltpu.async_copy(pbuf.at[r], t_sh.at[col_v.at[slot, b]],
                                 sem_s, add=True)
            return carry
        lax.fori_loop(0, nsb, sb_body, 0)
        drain_scatter((SB - 1) % 2)
        plsc.subcore_barrier()

        pltpu.sync_copy(t_sh.at[pl.ds(s * rpt, rpt)],
                        out_hbm.at[c, pl.ds(s * rpt, rpt)])

    return ek(p, q, row3, col3)


@functools.partial(jax.jit, static_argnums=(0, 1, 2))
def _sc_degree(n2, d, steps, col3):
    kd = CH // L
    rpt = n2 // NS
    nzb = rpt // CH
    nsb = steps // SB

    @functools.partial(
        pl.kernel,
        out_type=jax.ShapeDtypeStruct((NC, n2), jnp.float32),
        mesh=_sc_mesh(),
        scratch_types=[
            pltpu.VMEM((2, SB, CH), jnp.int32),
            pltpu.VMEM((CH,), jnp.float32),
            pltpu.VMEM_SHARED((n2,), jnp.float32),
            pltpu.SemaphoreType.DMA,
            pltpu.SemaphoreType.DMA,
        ],
    )
    def dk(col_hbm, out_hbm, col_v, buf, d_sh, sem_i, sem_s):
        c = lax.axis_index("c")
        s = lax.axis_index("s")
        w = c * NS + s

        def fill(val):
            for k in range(kd):
                buf[pl.ds(k * L, L)] = jnp.full((L,), val, jnp.float32)

        fill(0.0)

        def zt(i, carry):
            pltpu.sync_copy(buf, d_sh.at[pl.ds(s * rpt + i * CH, CH)])
            return carry
        lax.fori_loop(0, nzb, zt, 0)
        fill(1.0)
        plsc.subcore_barrier()

        def issue_idx(slot, sb):
            pltpu.async_copy(col_hbm.at[w, pl.ds(sb * SB, SB)],
                             col_v.at[slot], sem_i)

        def wait_idx(slot):
            pltpu.make_async_copy(col_hbm.at[w, pl.ds(0, SB)],
                                  col_v.at[slot], sem_i).wait()

        issue_idx(0, 0)
        wait_idx(0)

        def sb_body(sb, carry):
            slot = sb % 2
            nslot = 1 - slot

            @pl.when(sb + 1 < nsb)
            def _prefetch_idx():
                issue_idx(nslot, sb + 1)

            for b in range(SB):
                if b >= 2:
                    pltpu.make_async_copy(
                        buf, d_sh.at[col_v.at[0, 0]], sem_s).wait()
                else:
                    @pl.when(sb >= 1)
                    def _drain0():
                        pltpu.make_async_copy(
                            buf, d_sh.at[col_v.at[0, 0]], sem_s).wait()
                if b == SB - 1:
                    @pl.when(sb + 1 < nsb)
                    def _wait_next_idx():
                        wait_idx(nslot)
                pltpu.async_copy(buf, d_sh.at[col_v.at[slot, b]], sem_s,
                                 add=True)
            return carry
        lax.fori_loop(0, nsb, sb_body, 0)
        pltpu.make_async_copy(buf, d_sh.at[col_v.at[0, 0]], sem_s).wait()
        pltpu.make_async_copy(buf, d_sh.at[col_v.at[0, 0]], sem_s).wait()
        plsc.subcore_barrier()

        pltpu.sync_copy(d_sh.at[pl.ds(s * rpt, rpt)],
                        out_hbm.at[c, pl.ds(s * rpt, rpt)])

    return dk(col3)


def kernel(x, edge_index, params):
    n, d = x.shape
    e = edge_index.shape[1]
    nw = NC * NS

    n2 = _cdiv(n, NS * CH) * NS * CH
    steps = _cdiv(_cdiv(e, nw), CH)
    steps = _cdiv(steps, SB) * SB
    e_pad = nw * steps * CH
    if e_pad > e and n2 == n:
        n2 += NS * CH
    br = 1024 if n2 % 1024 == 0 else n2 // NS

    x_pad = jnp.zeros((n2, d), jnp.float32).at[:n].set(x)

    n_spare = max(n2 - n, 1)
    pad_idx = (jnp.arange(e_pad - e, dtype=jnp.int32) % n_spare) + n
    row3 = jnp.concatenate([edge_index[0], pad_idx]).reshape(nw, steps, CH)
    col3 = jnp.concatenate([edge_index[1], pad_idx]).reshape(nw, steps, CH)

    prm = params
    lp = prm["layers"]

    def b_(v):
        return v.reshape(1, d)

    def msg_w(li):
        return li["mW1"][:d], li["mW1"][d:], b_(li["mb1"])

    dg = _sc_degree(n2, d, steps, col3).reshape(NC, n2, 1)

    h, p, q = _tc_prep(n2, d, br, x_pad, prm["Wi"], b_(prm["bi"]),
                       *msg_w(lp[0]))

    for i in range(len(lp)):
        t2 = _sc_edge_pass(n2, d, steps, p, q, row3, col3)
        li = lp[i]
        common = (h, t2, dg, li["mW2"], b_(li["mb2"]),
                  li["uW1"][:d], li["uW1"][d:], b_(li["ub1"]),
                  li["uW2"], b_(li["ub2"]))
        if i + 1 < len(lp):
            h, p, q = _tc_update_mid(n2, d, br, *common, *msg_w(lp[i + 1]))
        else:
            out = _tc_update_final(n2, d, br, *common,
                                   prm["Wo"], b_(prm["bo"]))
    return out[:n]
